# baseline (speedup 1.0000x reference)
import functools

import jax
import jax.numpy as jnp
from jax import lax
from jax.experimental import pallas as pl
from jax.experimental.pallas import tpu as pltpu

_NUM_CLASSES = 20
_NEG_POS_RATIO = 3
_ALPHA = 0.2

_INT_MIN = -(2**31)
_INT_MAX = 2**31 - 1


def _focal_of_bg(bg):
    p = jnp.exp(-bg)
    return _ALPHA * (1.0 - p) * (1.0 - p) * bg


def _ssd_kernel(pt_ref, lab_ref, g0_ref, g1_ref, g2_ref, g3_ref, out_ref,
                keys_ref, acc_ref, kacc_ref, *, bp):
    i = pl.program_id(0)
    nb = pl.num_programs(0)
    B = lab_ref.shape[0]
    lb = lab_ref[...]

    m = pt_ref[4]
    for c in range(5, 25):
        m = jnp.maximum(m, pt_ref[c])
    s = jnp.zeros_like(m)
    cel = jnp.zeros_like(m)
    for c in range(4, 25):
        pc = pt_ref[c]
        s = s + jnp.exp(pc - m)
        cel = jnp.where(lb == (c - 4), pc, cel)
    lse = m + jnp.log(s)
    bg = lse - pt_ref[4]
    ce = lse - cel
    pt = jnp.exp(-ce)
    focal = _ALPHA * (1.0 - pt) * (1.0 - pt) * ce
    pos = lb > 0

    bits = lax.bitcast_convert_type(bg, jnp.int32)
    ikey = jnp.where(bits >= 0, bits, bits ^ jnp.int32(0x7FFFFFFF))
    ikey = jnp.where(pos, _INT_MIN, ikey)
    keys_ref[:, pl.ds(i * bp, bp)] = ikey

    def sl1(d):
        ad = jnp.abs(d)
        return jnp.where(ad < 1.0, 0.5 * d * d, ad - 0.5)

    box_row = (sl1(1.0 / (1.0 + jnp.exp(-pt_ref[0])) - g0_ref[...])
               + sl1(1.0 / (1.0 + jnp.exp(-pt_ref[1])) - g1_ref[...])
               + sl1(pt_ref[2] - g2_ref[...])
               + sl1(pt_ref[3] - g3_ref[...]))

    posf = pos.astype(jnp.float32)
    npos_blk = jnp.sum(posf, axis=1)
    fpos_blk = jnp.sum(jnp.where(pos, focal, 0.0), axis=1)
    box_blk = jnp.sum(jnp.where(pos, box_row, 0.0), axis=1)
    kmin_blk = jnp.min(jnp.where(pos, _INT_MAX, ikey), axis=1)
    kmax_blk = jnp.max(ikey, axis=1)
    lane = lax.broadcasted_iota(jnp.int32, (B, 128), 1)
    upd = (jnp.where(lane == 0, npos_blk[:, None], 0.0)
           + jnp.where(lane == 1, fpos_blk[:, None], 0.0)
           + jnp.where(lane == 2, box_blk[:, None], 0.0))
    kupd = jnp.where(lane == 0, kmin_blk[:, None],
                     jnp.where(lane == 1, kmax_blk[:, None], 0))

    @pl.when(i == 0)
    def _():
        acc_ref[...] = upd
        kacc_ref[...] = kupd

    @pl.when(i > 0)
    def _():
        acc_ref[...] = acc_ref[...] + upd
        old = kacc_ref[...]
        kacc_ref[...] = jnp.where(lane == 0, jnp.minimum(old, kupd),
                                  jnp.maximum(old, kupd))

    @pl.when(i == nb - 1)
    def _():
        P = nb * bp
        acc = acc_ref[...]
        npos = acc[:, 0:1]
        nposi = npos.astype(jnp.int32)
        k = jnp.minimum(nposi * _NEG_POS_RATIO, P - nposi)

        def count_gt(x):
            kk = keys_ref[...]
            return jnp.sum((kk > x).astype(jnp.int32), axis=1, keepdims=True)

        def sbody(lh):
            lo, hi = lh
            mid = (lo >> 1) + (hi >> 1) + (lo & hi & 1)
            ge = count_gt(mid) >= k
            return (jnp.where(ge, mid + 1, lo), jnp.where(ge, hi, mid))

        kmin = kacc_ref[:, 0:1]
        kmax = kacc_ref[:, 1:2]
        lo = jnp.where(k > 0, jnp.minimum(kmin, kmax), kmax)
        hi = kmax
        lo, hi = lax.while_loop(lambda lh: jnp.any(lh[0] < lh[1]),
                                sbody, (lo, hi))
        thr = lo

        def key_to_bg(x):
            bits_ = jnp.where(x >= 0, x, x ^ jnp.int32(0x7FFFFFFF))
            return lax.bitcast_convert_type(bits_, jnp.float32)

        kk = keys_ref[...]
        gt = kk > thr
        cnt = jnp.sum(gt.astype(jnp.int32), axis=1, keepdims=True)
        fstrict = jnp.sum(jnp.where(gt, _focal_of_bg(key_to_bg(kk)), 0.0),
                          axis=1, keepdims=True)
        need = k - cnt
        fthr = _focal_of_bg(key_to_bg(thr))
        fneg = (fstrict + jnp.where(need > 0, need.astype(jnp.float32) * fthr,
                                    0.0))[:, 0]
        nposc = jnp.maximum(acc[:, 0], 1.0)
        clf = (acc[:, 1] + fneg) / nposc
        boxl = acc[:, 2] / nposc
        loss_box = jnp.sum(boxl)
        loss_clf = jnp.sum(clf)
        olane = lax.broadcasted_iota(jnp.int32, (1, 128), 1)
        out_ref[...] = jnp.where(olane == 0, loss_box,
                                 jnp.where(olane == 1, loss_clf, 0.0))


def kernel(preds, gt_locations, labels):
    B, P, D = preds.shape
    bp = 8192
    nb = P // bp
    labels = labels.astype(jnp.int32)
    pt = jnp.transpose(preds, (2, 0, 1))
    gtf = jnp.transpose(gt_locations, (0, 2, 1)).reshape(B, 4 * P)
    gspec = [pl.BlockSpec((B, bp), functools.partial(
        lambda c, i: (0, c * (P // bp) + i), c)) for c in range(4)]
    out = pl.pallas_call(
        functools.partial(_ssd_kernel, bp=bp),
        grid=(nb,),
        in_specs=[
            pl.BlockSpec((D, B, bp), lambda i: (0, 0, i)),
            pl.BlockSpec((B, bp), lambda i: (0, i)),
        ] + gspec,
        out_specs=pl.BlockSpec((1, 128), lambda i: (0, 0)),
        out_shape=jax.ShapeDtypeStruct((1, 128), jnp.float32),
        scratch_shapes=[
            pltpu.VMEM((B, P), jnp.int32),
            pltpu.VMEM((B, 128), jnp.float32),
            pltpu.VMEM((B, 128), jnp.int32),
        ],
    )(pt, labels, *([gtf] * 4))
    return out[0, :2]


# speedup vs baseline: 1.1830x; 1.1830x over previous
import functools

import jax
import jax.numpy as jnp
from jax import lax
from jax.experimental import pallas as pl
from jax.experimental.pallas import tpu as pltpu

_NUM_CLASSES = 20
_NEG_POS_RATIO = 3
_ALPHA = 0.2

_INT_MIN = -(2**31)
_INT_MAX = 2**31 - 1


def _focal_of_bg(bg):
    p = jnp.exp(-bg)
    return _ALPHA * (1.0 - p) * (1.0 - p) * bg


def _ssd_kernel(pt_ref, lab_ref, g0_ref, g1_ref, g2_ref, g3_ref, out_ref,
                keys_ref, acc_ref, kacc_ref, *, bp):
    i = pl.program_id(0)
    nb = pl.num_programs(0)
    B = lab_ref.shape[0]
    lb = lab_ref[...]

    m = pt_ref[4]
    for c in range(5, 25):
        m = jnp.maximum(m, pt_ref[c])
    s = jnp.zeros_like(m)
    cel = jnp.zeros_like(m)
    for c in range(4, 25):
        pc = pt_ref[c]
        s = s + jnp.exp(pc - m)
        cel = jnp.where(lb == (c - 4), pc, cel)
    lse = m + jnp.log(s)
    bg = lse - pt_ref[4]
    ce = lse - cel
    pt = jnp.exp(-ce)
    focal = _ALPHA * (1.0 - pt) * (1.0 - pt) * ce
    pos = lb > 0

    bits = lax.bitcast_convert_type(bg, jnp.int32)
    ikey = jnp.where(bits >= 0, bits, bits ^ jnp.int32(0x7FFFFFFF))
    ikey = jnp.where(pos, _INT_MIN, ikey)
    keys_ref[:, pl.ds(i * bp, bp)] = ikey

    def sl1(d):
        ad = jnp.abs(d)
        return jnp.where(ad < 1.0, 0.5 * d * d, ad - 0.5)

    box_row = (sl1(1.0 / (1.0 + jnp.exp(-pt_ref[0])) - g0_ref[...])
               + sl1(1.0 / (1.0 + jnp.exp(-pt_ref[1])) - g1_ref[...])
               + sl1(pt_ref[2] - g2_ref[...])
               + sl1(pt_ref[3] - g3_ref[...]))

    posf = pos.astype(jnp.float32)
    npos_blk = jnp.sum(posf, axis=1)
    fpos_blk = jnp.sum(jnp.where(pos, focal, 0.0), axis=1)
    box_blk = jnp.sum(jnp.where(pos, box_row, 0.0), axis=1)
    kmin_blk = jnp.min(jnp.where(pos, _INT_MAX, ikey), axis=1)
    kmax_blk = jnp.max(ikey, axis=1)
    lane = lax.broadcasted_iota(jnp.int32, (B, 128), 1)
    upd = (jnp.where(lane == 0, npos_blk[:, None], 0.0)
           + jnp.where(lane == 1, fpos_blk[:, None], 0.0)
           + jnp.where(lane == 2, box_blk[:, None], 0.0))
    kupd = jnp.where(lane == 0, kmin_blk[:, None],
                     jnp.where(lane == 1, kmax_blk[:, None], 0))

    @pl.when(i == 0)
    def _():
        acc_ref[...] = upd
        kacc_ref[...] = kupd

    @pl.when(i > 0)
    def _():
        acc_ref[...] = acc_ref[...] + upd
        old = kacc_ref[...]
        kacc_ref[...] = jnp.where(lane == 0, jnp.minimum(old, kupd),
                                  jnp.maximum(old, kupd))

    @pl.when(i == nb - 1)
    def _():
        P = nb * bp
        acc = acc_ref[...]
        npos = acc[:, 0:1]
        nposi = npos.astype(jnp.int32)
        k = jnp.minimum(nposi * _NEG_POS_RATIO, P - nposi)

        def sbody(lh):
            lo, hi = lh
            d = hi - lo
            m1 = lo + (d >> 2)
            m2 = lo + (d >> 1)
            m3 = m2 + (d >> 2)
            kk = keys_ref[...]
            c1 = jnp.sum((kk > m1).astype(jnp.int32), axis=1, keepdims=True)
            c2 = jnp.sum((kk > m2).astype(jnp.int32), axis=1, keepdims=True)
            c3 = jnp.sum((kk > m3).astype(jnp.int32), axis=1, keepdims=True)
            ge1 = c1 >= k
            ge2 = c2 >= k
            ge3 = c3 >= k
            nlo = jnp.where(ge3, m3 + 1,
                            jnp.where(ge2, m2 + 1,
                                      jnp.where(ge1, m1 + 1, lo)))
            nhi = jnp.where(ge3, hi,
                            jnp.where(ge2, m3, jnp.where(ge1, m2, m1)))
            return (nlo, nhi)

        kmin = kacc_ref[:, 0:1]
        kmax = kacc_ref[:, 1:2]
        lo = jnp.where(k > 0, jnp.minimum(kmin, kmax), kmax)
        hi = kmax
        lo, hi = lax.while_loop(lambda lh: jnp.any(lh[0] < lh[1]),
                                sbody, (lo, hi))
        thr = lo

        def key_to_bg(x):
            bits_ = jnp.where(x >= 0, x, x ^ jnp.int32(0x7FFFFFFF))
            return lax.bitcast_convert_type(bits_, jnp.float32)

        kk = keys_ref[...]
        gt = kk > thr
        cnt = jnp.sum(gt.astype(jnp.int32), axis=1, keepdims=True)
        fstrict = jnp.sum(jnp.where(gt, _focal_of_bg(key_to_bg(kk)), 0.0),
                          axis=1, keepdims=True)
        need = k - cnt
        fthr = _focal_of_bg(key_to_bg(thr))
        fneg = (fstrict + jnp.where(need > 0, need.astype(jnp.float32) * fthr,
                                    0.0))[:, 0]
        nposc = jnp.maximum(acc[:, 0], 1.0)
        clf = (acc[:, 1] + fneg) / nposc
        boxl = acc[:, 2] / nposc
        loss_box = jnp.sum(boxl)
        loss_clf = jnp.sum(clf)
        olane = lax.broadcasted_iota(jnp.int32, (1, 128), 1)
        out_ref[...] = jnp.where(olane == 0, loss_box,
                                 jnp.where(olane == 1, loss_clf, 0.0))


def kernel(preds, gt_locations, labels):
    B, P, D = preds.shape
    bp = 4096
    nb = P // bp
    labels = labels.astype(jnp.int32)
    pt = jnp.transpose(preds, (2, 0, 1))
    gtf = jnp.transpose(gt_locations, (0, 2, 1)).reshape(B, 4 * P)
    gspec = [pl.BlockSpec((B, bp), functools.partial(
        lambda c, i: (0, c * (P // bp) + i), c)) for c in range(4)]
    out = pl.pallas_call(
        functools.partial(_ssd_kernel, bp=bp),
        grid=(nb,),
        in_specs=[
            pl.BlockSpec((D, B, bp), lambda i: (0, 0, i)),
            pl.BlockSpec((B, bp), lambda i: (0, i)),
        ] + gspec,
        out_specs=pl.BlockSpec((1, 128), lambda i: (0, 0)),
        out_shape=jax.ShapeDtypeStruct((1, 128), jnp.float32),
        scratch_shapes=[
            pltpu.VMEM((B, P), jnp.int32),
            pltpu.VMEM((B, 128), jnp.float32),
            pltpu.VMEM((B, 128), jnp.int32),
        ],
    )(pt, labels, *([gtf] * 4))
    return out[0, :2]
